# baseline (speedup 1.0000x reference)
import jax
import jax.numpy as jnp
from jax import lax
from jax.experimental import pallas as pl
from jax.experimental.pallas import tpu as pltpu
from jax.experimental.pallas import tpu_sc as plsc

N = 10000
E = 320000
D = 129
L = 16
DP = 160
DH = DP // 2
NBH = DH // L
NC = 2
NS = 16
C = 80
NCHB = 5
BS = C * NCHB
ER = E // C
RPT2 = ER // NS
NBATCH = RPT2 // NCHB
RPT = 624
RF = RPT // C
RR = RPT - RF * C
TAILR = N - NS * RPT

KLUT = 4096
VMAX = 20.0
DLUT = 2.0 * VMAX / KLUT
SLUT = KLUT / (2.0 * VMAX)
KR = KLUT // 128


def _lut_body(w2_ref, o_ref):
    r = lax.broadcasted_iota(jnp.int32, (KR, 128), 0)
    c = lax.broadcasted_iota(jnp.int32, (KR, 128), 1)
    v = (r * 128 + c).astype(jnp.float32) * DLUT - VMAX
    w2 = w2_ref[0, 0]
    o_ref[...] = jnp.tanh(w2 * jnp.tanh(0.5 * v))


def _lut_call(w2s):
    return pl.pallas_call(
        _lut_body,
        in_specs=[pl.BlockSpec(memory_space=pltpu.SMEM)],
        out_shape=jax.ShapeDtypeStruct((KR, 128), jnp.float32),
    )(w2s)


def _sc_body(f2_hbm, src_hbm, tgt_hbm, a_hbm, d_hbm, w1_hbm, lut_hbm,
             agg0_hbm, agg1_hbm,
             src_b, tgt_b, sidx_b, tidx_b, a_b, d_b, ua_b, w1_v, lut_v,
             rows_s0, rows_s1, rows_t0, rows_t1, msg0, msg1, agg_sh,
             sem_s, sem_t, ssem0, ssem1):
    rows_s = (rows_s0, rows_s1)
    rows_t = (rows_t0, rows_t1)
    msgs = (msg0, msg1)
    ssems = (ssem0, ssem1)
    msg = msg0
    cid = lax.axis_index("c")
    sid = lax.axis_index("s")

    pltpu.sync_copy(w1_hbm, w1_v)
    pltpu.sync_copy(lut_hbm, lut_v)
    w1r = w1_v[...]

    zero = jnp.zeros((L,), jnp.float32)

    def zrow(r, carry):
        for b in range(NBH):
            msg[r, pl.ds(b * L, L)] = zero
        return carry

    lax.fori_loop(0, C, zrow, 0)

    row0 = pl.multiple_of(sid * RPT, 8)

    def zcp(k, carry):
        pltpu.sync_copy(msg,
                        agg_sh.at[pl.ds(pl.multiple_of(row0 + k * C, 8), C)])
        return carry

    lax.fori_loop(0, RF, zcp, 0)
    pltpu.sync_copy(msg.at[pl.ds(0, RR)],
                    agg_sh.at[pl.ds(pl.multiple_of(row0 + RF * C, 8), RR)])

    @pl.when(sid == NS - 1)
    def _():
        pltpu.sync_copy(msg.at[pl.ds(0, TAILR)],
                        agg_sh.at[pl.ds(N - TAILR, TAILR)])

    plsc.subcore_barrier()

    lane = lax.iota(jnp.int32, L)
    cidv = jnp.full((L,), cid, jnp.int32)
    md = jnp.logical_and(lane == 1, cidv == 1)
    mc = jnp.logical_and(lane == 2, cidv == 1)
    vclmp = jnp.full((L,), VMAX - DLUT, jnp.float32)
    vlo = jnp.full((L,), -VMAX, jnp.float32)
    lane8 = jnp.bitwise_and(lane, 7)

    def batch(bi, carry):
        rb = sid * RPT2 + bi * NCHB
        pltpu.sync_copy(src_hbm.at[pl.ds(rb, NCHB)], src_b)
        pltpu.sync_copy(tgt_hbm.at[pl.ds(rb, NCHB)], tgt_b)
        pltpu.sync_copy(a_hbm.at[pl.ds(rb, NCHB)], a_b)
        pltpu.sync_copy(d_hbm.at[pl.ds(rb, NCHB)], d_b.at[pl.ds(0, NCHB)])
        for j in range(NCHB):
            for i in range(C // L):
                sl = pl.ds(i * L, L)
                ua_b[j, sl] = (w1r + w1r) * a_b[j, sl]
                s_ = src_b[j, sl]
                t_ = tgt_b[j, sl]
                sidx_b[j, sl] = s_ + s_ + cidv
                tidx_b[j, sl] = t_ + t_ + cidv

        gat = [None, None]
        sca = [None, None]
        gat[0] = (pltpu.async_copy(f2_hbm.at[sidx_b.at[0]], rows_s[0], sem_s),
                  pltpu.async_copy(f2_hbm.at[tidx_b.at[0]], rows_t[0], sem_t))
        for j in range(NCHB):
            p = j % 2
            gat[p][0].wait()
            gat[p][1].wait()
            if j + 1 < NCHB:
                q = 1 - p
                gat[q] = (
                    pltpu.async_copy(f2_hbm.at[sidx_b.at[j + 1]],
                                     rows_s[q], sem_s),
                    pltpu.async_copy(f2_hbm.at[tidx_b.at[j + 1]],
                                     rows_t[q], sem_t))
            if sca[p] is not None:
                sca[p].wait()
            rs, rt, mg = rows_s[p], rows_t[p], msgs[p]

            def edge(e, ecarry, j=j, rs=rs, rt=rt, mg=mg):
                ua = jnp.full((L,), ua_b[j, pl.ds(e, L)][0], jnp.float32)
                for b in range(NBH):
                    xs = rs[e, pl.ds(b * L, L)]
                    xt = rt[e, pl.ds(b * L, L)]
                    ps = jnp.minimum(jnp.maximum(ua * xs, vlo), vclmp) \
                        * SLUT + (KLUT / 2.0 + 0.5)
                    pt = jnp.minimum(jnp.maximum(ua * xt, vlo), vclmp) \
                        * SLUT + (KLUT / 2.0 + 0.5)
                    i8s = ps.astype(jnp.int32) * 8 + lane8
                    i8t = pt.astype(jnp.int32) * 8 + lane8
                    m = (plsc.load_gather(lut_v, [i8s])
                         + plsc.load_gather(lut_v, [i8t]))
                    if b == 3:
                        de = jnp.full((L,), d_b[j, pl.ds(e, L)][0],
                                      jnp.float32)
                        m = jnp.where(md, de, m)
                        m = jnp.where(mc, jnp.float32(1.0), m)
                    mg[e, pl.ds(b * L, L)] = m
                return ecarry

            lax.fori_loop(0, C, edge, 0)
            sca[p] = pltpu.async_copy(mg, agg_sh.at[tgt_b.at[j]], ssems[p],
                                      add=True)
        sca[0].wait()
        sca[1].wait()
        return carry

    lax.fori_loop(0, NBATCH, batch, 0)
    plsc.subcore_barrier()

    @pl.when(cid == 0)
    def _():
        def wout(k, carry):
            r = pl.multiple_of(row0 + k * C, 8)
            pltpu.sync_copy(agg_sh.at[pl.ds(r, C)], agg0_hbm.at[pl.ds(r, C)])
            return carry

        lax.fori_loop(0, RF, wout, 0)
        rl = pl.multiple_of(row0 + RF * C, 8)
        pltpu.sync_copy(agg_sh.at[pl.ds(rl, RR)], agg0_hbm.at[pl.ds(rl, RR)])

        @pl.when(sid == NS - 1)
        def _():
            pltpu.sync_copy(agg_sh.at[pl.ds(N - TAILR, TAILR)],
                            agg0_hbm.at[pl.ds(N - TAILR, TAILR)])

    @pl.when(cid == 1)
    def _():
        def wout(k, carry):
            r = pl.multiple_of(row0 + k * C, 8)
            pltpu.sync_copy(agg_sh.at[pl.ds(r, C)], agg1_hbm.at[pl.ds(r, C)])
            return carry

        lax.fori_loop(0, RF, wout, 0)
        rl = pl.multiple_of(row0 + RF * C, 8)
        pltpu.sync_copy(agg_sh.at[pl.ds(rl, RR)], agg1_hbm.at[pl.ds(rl, RR)])

        @pl.when(sid == NS - 1)
        def _():
            pltpu.sync_copy(agg_sh.at[pl.ds(N - TAILR, TAILR)],
                            agg1_hbm.at[pl.ds(N - TAILR, TAILR)])


_sc_call = pl.kernel(
    _sc_body,
    out_type=(jax.ShapeDtypeStruct((N, DH), jnp.float32),
              jax.ShapeDtypeStruct((N, DH), jnp.float32)),
    mesh=plsc.VectorSubcoreMesh(core_axis_name="c", subcore_axis_name="s"),
    compiler_params=pltpu.CompilerParams(use_tc_tiling_on_sc=False,
                                         needs_layout_passes=False),
    scratch_types=[
        pltpu.VMEM((NCHB, C), jnp.int32),
        pltpu.VMEM((NCHB, C), jnp.int32),
        pltpu.VMEM((NCHB, C), jnp.int32),
        pltpu.VMEM((NCHB, C), jnp.int32),
        pltpu.VMEM((NCHB, C), jnp.float32),
        pltpu.VMEM((NCHB + 1, C), jnp.float32),
        pltpu.VMEM((NCHB + 1, C), jnp.float32),
        pltpu.VMEM((L,), jnp.float32),
        pltpu.VMEM((KLUT * 8,), jnp.float32),
        pltpu.VMEM((C, DH), jnp.float32),
        pltpu.VMEM((C, DH), jnp.float32),
        pltpu.VMEM((C, DH), jnp.float32),
        pltpu.VMEM((C, DH), jnp.float32),
        pltpu.VMEM((C, DH), jnp.float32),
        pltpu.VMEM((C, DH), jnp.float32),
        pltpu.VMEM_SHARED((N, DH), jnp.float32),
        pltpu.SemaphoreType.DMA,
        pltpu.SemaphoreType.DMA,
        pltpu.SemaphoreType.DMA,
        pltpu.SemaphoreType.DMA,
    ],
)


BROWS = 2000


def _tc_body(f_ref, a0_ref, a1_ref, W1_ref, b1_ref, W2_ref, b2_ref,
             W3_ref, b3_ref, o_ref):
    agg = jnp.concatenate([a0_ref[...], a1_ref[...]], axis=1)
    col = lax.broadcasted_iota(jnp.int32, agg.shape, 1)
    aggm = jnp.where(col < D, agg, 0.0)
    nd = agg[:, D:D + 1]
    ncnt = agg[:, D + 1:D + 2]
    f = f_ref[...]
    f_inv = jnp.sqrt(jnp.sum(f * f, axis=1, keepdims=True))
    msg_inv = jnp.sqrt(jnp.sum(aggm * aggm, axis=1, keepdims=True))
    avg = nd / (ncnt + 1e-8)
    psi = jnp.concatenate([f_inv, msg_inv, avg], axis=1)
    h = jax.nn.relu(jnp.dot(psi, W1_ref[...].T,
                            preferred_element_type=jnp.float32) + b1_ref[...])
    h = jax.nn.relu(jnp.dot(h, W2_ref[...].T,
                            preferred_element_type=jnp.float32) + b2_ref[...])
    gate = jax.nn.sigmoid(jnp.sum(h * W3_ref[...], axis=1, keepdims=True)
                          + b3_ref[0, 0])
    o_ref[...] = f + gate * aggm


def _tc_call(fpad, a0, a1, W1, b1, W2, b2, W3, b3):
    full = lambda shape: pl.BlockSpec(shape, lambda i: (0, 0))
    return pl.pallas_call(
        _tc_body,
        grid=(N // BROWS,),
        in_specs=[
            pl.BlockSpec((BROWS, DP), lambda i: (i, 0)),
            pl.BlockSpec((BROWS, DH), lambda i: (i, 0)),
            pl.BlockSpec((BROWS, DH), lambda i: (i, 0)),
            full((64, 3)), full((1, 64)),
            full((32, 64)), full((1, 32)),
            full((1, 32)),
            pl.BlockSpec(memory_space=pltpu.SMEM),
        ],
        out_specs=pl.BlockSpec((BROWS, DP), lambda i: (i, 0)),
        out_shape=jax.ShapeDtypeStruct((N, DP), jnp.float32),
    )(fpad, a0, a1, W1, b1, W2, b2, W3, b3)


@jax.jit
def kernel(edge_index, f, d, a, w1, w2, W1, b1, W2, b2, W3, b3):
    src = edge_index[0].astype(jnp.int32).reshape(ER, C)
    tgt = edge_index[1].astype(jnp.int32).reshape(ER, C)
    fpad = jnp.pad(f, ((0, 0), (0, DP - D)))
    f2 = fpad.reshape(N, 2, DH).reshape(2 * N, DH)
    a2 = a[:, 0].reshape(ER, C)
    d2 = d[:, 0].reshape(ER, C)
    w1b = jnp.full((L,), w1[0], jnp.float32)
    lut = jnp.repeat(_lut_call(w2.reshape(1, 1)).reshape(KLUT), 8)
    agg0, agg1 = _sc_call(f2, src, tgt, a2, d2, w1b, lut)
    outp = _tc_call(fpad, agg0, agg1, W1, b1.reshape(1, 64),
                    W2, b2.reshape(1, 32), W3, b3.reshape(1, 1))
    return outp[:, :D]


# speedup vs baseline: 1.8819x; 1.8819x over previous
import jax
import jax.numpy as jnp
from jax import lax
from jax.experimental import pallas as pl
from jax.experimental.pallas import tpu as pltpu
from jax.experimental.pallas import tpu_sc as plsc

N = 10000
E = 320000
D = 129
L = 16
DP = 160
DH = DP // 2
NBH = DH // L
NC = 2
NS = 16
C = 80
NCHB = 5
BS = C * NCHB
ER = E // C
RPT2 = ER // NS
NBATCH = RPT2 // NCHB
RPT = 624
RF = RPT // C
RR = RPT - RF * C
TAILR = N - NS * RPT

KLUT = 4096
VMAX = 20.0
DLUT = 2.0 * VMAX / KLUT
SLUT = KLUT / (2.0 * VMAX)
KR = KLUT // 128


def _lut_body(w2_ref, o_ref):
    r = lax.broadcasted_iota(jnp.int32, (KR, 128), 0)
    c = lax.broadcasted_iota(jnp.int32, (KR, 128), 1)
    v = (r * 128 + c).astype(jnp.float32) * DLUT - VMAX
    w2 = w2_ref[0, 0]
    o_ref[...] = jnp.tanh(w2 * jnp.tanh(0.5 * v))


def _lut_call(w2s):
    return pl.pallas_call(
        _lut_body,
        in_specs=[pl.BlockSpec(memory_space=pltpu.SMEM)],
        out_shape=jax.ShapeDtypeStruct((KR, 128), jnp.float32),
    )(w2s)


def _sc_body(f2_hbm, src_hbm, tgt_hbm, a_hbm, d_hbm, w1_hbm, lut_hbm,
             agg0_hbm, agg1_hbm,
             src_b, tgt_b, sidx_b, tidx_b, a_b, d_b, ua_b, w1_v, lut_v,
             rows_s0, rows_s1, rows_t0, rows_t1, msg0, msg1, agg_sh,
             sem_s, sem_t, ssem0, ssem1):
    rows_s = (rows_s0, rows_s1)
    rows_t = (rows_t0, rows_t1)
    msgs = (msg0, msg1)
    ssems = (ssem0, ssem1)
    msg = msg0
    cid = lax.axis_index("c")
    sid = lax.axis_index("s")

    pltpu.sync_copy(w1_hbm, w1_v)
    pltpu.sync_copy(lut_hbm, lut_v)
    w1r = w1_v[...]

    zero = jnp.zeros((L,), jnp.float32)

    def zrow(r, carry):
        for b in range(NBH):
            msg[r, pl.ds(b * L, L)] = zero
        return carry

    lax.fori_loop(0, C, zrow, 0)

    row0 = pl.multiple_of(sid * RPT, 8)

    def zcp(k, carry):
        pltpu.sync_copy(msg,
                        agg_sh.at[pl.ds(pl.multiple_of(row0 + k * C, 8), C)])
        return carry

    lax.fori_loop(0, RF, zcp, 0)
    pltpu.sync_copy(msg.at[pl.ds(0, RR)],
                    agg_sh.at[pl.ds(pl.multiple_of(row0 + RF * C, 8), RR)])

    @pl.when(sid == NS - 1)
    def _():
        pltpu.sync_copy(msg.at[pl.ds(0, TAILR)],
                        agg_sh.at[pl.ds(N - TAILR, TAILR)])

    plsc.subcore_barrier()

    lane = lax.iota(jnp.int32, L)
    cidv = jnp.full((L,), cid, jnp.int32)
    md = jnp.logical_and(lane == 1, cidv == 1)
    mc = jnp.logical_and(lane == 2, cidv == 1)
    vclmp = jnp.full((L,), VMAX - DLUT, jnp.float32)
    vlo = jnp.full((L,), -VMAX, jnp.float32)
    lane8 = jnp.bitwise_and(lane, 7)

    def batch(bi, carry):
        rb = sid * RPT2 + bi * NCHB
        pltpu.sync_copy(src_hbm.at[pl.ds(rb, NCHB)], src_b)
        pltpu.sync_copy(tgt_hbm.at[pl.ds(rb, NCHB)], tgt_b)
        pltpu.sync_copy(a_hbm.at[pl.ds(rb, NCHB)], a_b)
        pltpu.sync_copy(d_hbm.at[pl.ds(rb, NCHB)], d_b.at[pl.ds(0, NCHB)])
        for j in range(NCHB):
            for i in range(C // L):
                sl = pl.ds(i * L, L)
                ua_b[j, sl] = (w1r + w1r) * a_b[j, sl]
                s_ = src_b[j, sl]
                t_ = tgt_b[j, sl]
                sidx_b[j, sl] = s_ + s_ + cidv
                tidx_b[j, sl] = t_ + t_ + cidv

        gat = [None, None]
        sca = [None, None]
        gat[0] = (pltpu.async_copy(f2_hbm.at[sidx_b.at[0]], rows_s[0], sem_s),
                  pltpu.async_copy(f2_hbm.at[tidx_b.at[0]], rows_t[0], sem_t))
        for j in range(NCHB):
            p = j % 2
            gat[p][0].wait()
            gat[p][1].wait()
            if j + 1 < NCHB:
                q = 1 - p
                gat[q] = (
                    pltpu.async_copy(f2_hbm.at[sidx_b.at[j + 1]],
                                     rows_s[q], sem_s),
                    pltpu.async_copy(f2_hbm.at[tidx_b.at[j + 1]],
                                     rows_t[q], sem_t))
            if sca[p] is not None:
                sca[p].wait()
            rs, rt, mg = rows_s[p], rows_t[p], msgs[p]

            def egroup(g, ecarry, j=j, rs=rs, rt=rt, mg=mg):
                e0 = g * L
                ua16 = ua_b[j, pl.ds(e0, L)]
                de16 = d_b[j, pl.ds(e0, L)]
                for jj in range(L):
                    ua = jnp.full((L,), ua16[jj], jnp.float32)
                    for b in range(NBH):
                        xs = rs[e0 + jj, pl.ds(b * L, L)]
                        xt = rt[e0 + jj, pl.ds(b * L, L)]
                        ps = jnp.minimum(jnp.maximum(ua * xs, vlo), vclmp) \
                            * SLUT + (KLUT / 2.0 + 0.5)
                        pt = jnp.minimum(jnp.maximum(ua * xt, vlo), vclmp) \
                            * SLUT + (KLUT / 2.0 + 0.5)
                        i8s = ps.astype(jnp.int32) * 8 + lane8
                        i8t = pt.astype(jnp.int32) * 8 + lane8
                        m = (plsc.load_gather(lut_v, [i8s])
                             + plsc.load_gather(lut_v, [i8t]))
                        if b == 3:
                            de = jnp.full((L,), de16[jj], jnp.float32)
                            m = jnp.where(md, de, m)
                            m = jnp.where(mc, jnp.float32(1.0), m)
                        mg[e0 + jj, pl.ds(b * L, L)] = m
                return ecarry

            lax.fori_loop(0, C // L, egroup, 0)
            sca[p] = pltpu.async_copy(mg, agg_sh.at[tgt_b.at[j]], ssems[p],
                                      add=True)
        sca[0].wait()
        sca[1].wait()
        return carry

    lax.fori_loop(0, NBATCH, batch, 0)
    plsc.subcore_barrier()

    @pl.when(cid == 0)
    def _():
        def wout(k, carry):
            r = pl.multiple_of(row0 + k * C, 8)
            pltpu.sync_copy(agg_sh.at[pl.ds(r, C)], agg0_hbm.at[pl.ds(r, C)])
            return carry

        lax.fori_loop(0, RF, wout, 0)
        rl = pl.multiple_of(row0 + RF * C, 8)
        pltpu.sync_copy(agg_sh.at[pl.ds(rl, RR)], agg0_hbm.at[pl.ds(rl, RR)])

        @pl.when(sid == NS - 1)
        def _():
            pltpu.sync_copy(agg_sh.at[pl.ds(N - TAILR, TAILR)],
                            agg0_hbm.at[pl.ds(N - TAILR, TAILR)])

    @pl.when(cid == 1)
    def _():
        def wout(k, carry):
            r = pl.multiple_of(row0 + k * C, 8)
            pltpu.sync_copy(agg_sh.at[pl.ds(r, C)], agg1_hbm.at[pl.ds(r, C)])
            return carry

        lax.fori_loop(0, RF, wout, 0)
        rl = pl.multiple_of(row0 + RF * C, 8)
        pltpu.sync_copy(agg_sh.at[pl.ds(rl, RR)], agg1_hbm.at[pl.ds(rl, RR)])

        @pl.when(sid == NS - 1)
        def _():
            pltpu.sync_copy(agg_sh.at[pl.ds(N - TAILR, TAILR)],
                            agg1_hbm.at[pl.ds(N - TAILR, TAILR)])


_sc_call = pl.kernel(
    _sc_body,
    out_type=(jax.ShapeDtypeStruct((N, DH), jnp.float32),
              jax.ShapeDtypeStruct((N, DH), jnp.float32)),
    mesh=plsc.VectorSubcoreMesh(core_axis_name="c", subcore_axis_name="s"),
    compiler_params=pltpu.CompilerParams(use_tc_tiling_on_sc=False,
                                         needs_layout_passes=False),
    scratch_types=[
        pltpu.VMEM((NCHB, C), jnp.int32),
        pltpu.VMEM((NCHB, C), jnp.int32),
        pltpu.VMEM((NCHB, C), jnp.int32),
        pltpu.VMEM((NCHB, C), jnp.int32),
        pltpu.VMEM((NCHB, C), jnp.float32),
        pltpu.VMEM((NCHB + 1, C), jnp.float32),
        pltpu.VMEM((NCHB + 1, C), jnp.float32),
        pltpu.VMEM((L,), jnp.float32),
        pltpu.VMEM((KLUT * 8,), jnp.float32),
        pltpu.VMEM((C, DH), jnp.float32),
        pltpu.VMEM((C, DH), jnp.float32),
        pltpu.VMEM((C, DH), jnp.float32),
        pltpu.VMEM((C, DH), jnp.float32),
        pltpu.VMEM((C, DH), jnp.float32),
        pltpu.VMEM((C, DH), jnp.float32),
        pltpu.VMEM_SHARED((N, DH), jnp.float32),
        pltpu.SemaphoreType.DMA,
        pltpu.SemaphoreType.DMA,
        pltpu.SemaphoreType.DMA,
        pltpu.SemaphoreType.DMA,
    ],
)


BROWS = 2000


def _tc_body(f_ref, a0_ref, a1_ref, W1_ref, b1_ref, W2_ref, b2_ref,
             W3_ref, b3_ref, o_ref):
    agg = jnp.concatenate([a0_ref[...], a1_ref[...]], axis=1)
    col = lax.broadcasted_iota(jnp.int32, agg.shape, 1)
    aggm = jnp.where(col < D, agg, 0.0)
    nd = agg[:, D:D + 1]
    ncnt = agg[:, D + 1:D + 2]
    f = f_ref[...]
    f_inv = jnp.sqrt(jnp.sum(f * f, axis=1, keepdims=True))
    msg_inv = jnp.sqrt(jnp.sum(aggm * aggm, axis=1, keepdims=True))
    avg = nd / (ncnt + 1e-8)
    psi = jnp.concatenate([f_inv, msg_inv, avg], axis=1)
    h = jax.nn.relu(jnp.dot(psi, W1_ref[...].T,
                            preferred_element_type=jnp.float32) + b1_ref[...])
    h = jax.nn.relu(jnp.dot(h, W2_ref[...].T,
                            preferred_element_type=jnp.float32) + b2_ref[...])
    gate = jax.nn.sigmoid(jnp.sum(h * W3_ref[...], axis=1, keepdims=True)
                          + b3_ref[0, 0])
    o_ref[...] = f + gate * aggm


def _tc_call(fpad, a0, a1, W1, b1, W2, b2, W3, b3):
    full = lambda shape: pl.BlockSpec(shape, lambda i: (0, 0))
    return pl.pallas_call(
        _tc_body,
        grid=(N // BROWS,),
        in_specs=[
            pl.BlockSpec((BROWS, DP), lambda i: (i, 0)),
            pl.BlockSpec((BROWS, DH), lambda i: (i, 0)),
            pl.BlockSpec((BROWS, DH), lambda i: (i, 0)),
            full((64, 3)), full((1, 64)),
            full((32, 64)), full((1, 32)),
            full((1, 32)),
            pl.BlockSpec(memory_space=pltpu.SMEM),
        ],
        out_specs=pl.BlockSpec((BROWS, DP), lambda i: (i, 0)),
        out_shape=jax.ShapeDtypeStruct((N, DP), jnp.float32),
    )(fpad, a0, a1, W1, b1, W2, b2, W3, b3)


@jax.jit
def kernel(edge_index, f, d, a, w1, w2, W1, b1, W2, b2, W3, b3):
    src = edge_index[0].astype(jnp.int32).reshape(ER, C)
    tgt = edge_index[1].astype(jnp.int32).reshape(ER, C)
    fpad = jnp.pad(f, ((0, 0), (0, DP - D)))
    f2 = fpad.reshape(N, 2, DH).reshape(2 * N, DH)
    a2 = a[:, 0].reshape(ER, C)
    d2 = d[:, 0].reshape(ER, C)
    w1b = jnp.full((L,), w1[0], jnp.float32)
    lut = jnp.repeat(_lut_call(w2.reshape(1, 1)).reshape(KLUT), 8)
    agg0, agg1 = _sc_call(f2, src, tgt, a2, d2, w1b, lut)
    outp = _tc_call(fpad, agg0, agg1, W1, b1.reshape(1, 64),
                    W2, b2.reshape(1, 32), W3, b3.reshape(1, 1))
    return outp[:, :D]


# speedup vs baseline: 1.9337x; 1.0275x over previous
import jax
import jax.numpy as jnp
from jax import lax
from jax.experimental import pallas as pl
from jax.experimental.pallas import tpu as pltpu
from jax.experimental.pallas import tpu_sc as plsc

N = 10000
E = 320000
D = 129
L = 16
DP = 160
DH = DP // 2
NBH = DH // L
NC = 2
NS = 16
C = 80
NCHB = 5
BS = C * NCHB
ER = E // C
RPT2 = ER // NS
NBATCH = RPT2 // NCHB
RPT = 624
RF = RPT // C
RR = RPT - RF * C
TAILR = N - NS * RPT

KLUT = 32768
VMAX = 20.0
DLUT = 2.0 * VMAX / KLUT
SLUT = KLUT / (2.0 * VMAX)
KR = KLUT // 128


def _lut_body(w2_ref, o_ref):
    r = lax.broadcasted_iota(jnp.int32, (KR, 128), 0)
    c = lax.broadcasted_iota(jnp.int32, (KR, 128), 1)
    v = (r * 128 + c).astype(jnp.float32) * DLUT - VMAX
    w2 = w2_ref[0, 0]
    o_ref[...] = jnp.tanh(w2 * jnp.tanh(0.5 * v))


def _lut_call(w2s):
    return pl.pallas_call(
        _lut_body,
        in_specs=[pl.BlockSpec(memory_space=pltpu.SMEM)],
        out_shape=jax.ShapeDtypeStruct((KR, 128), jnp.float32),
    )(w2s)


def _sc_body(f2_hbm, src_hbm, tgt_hbm, a_hbm, d_hbm, w1_hbm, lut_hbm,
             agg0_hbm, agg1_hbm,
             src_b, tgt_b, sidx_b, tidx_b, a_b, d_b, ua_b, w1_v, lut_v,
             rows_s0, rows_s1, rows_t0, rows_t1, msg0, msg1, agg_sh,
             sem_s, sem_t, ssem0, ssem1):
    rows_s = (rows_s0, rows_s1)
    rows_t = (rows_t0, rows_t1)
    msgs = (msg0, msg1)
    ssems = (ssem0, ssem1)
    msg = msg0
    cid = lax.axis_index("c")
    sid = lax.axis_index("s")

    pltpu.sync_copy(w1_hbm, w1_v)
    pltpu.sync_copy(lut_hbm, lut_v)
    w1r = w1_v[...]

    zero = jnp.zeros((L,), jnp.float32)

    def zrow(r, carry):
        for b in range(NBH):
            msg[r, pl.ds(b * L, L)] = zero
        return carry

    lax.fori_loop(0, C, zrow, 0)

    row0 = pl.multiple_of(sid * RPT, 8)

    def zcp(k, carry):
        pltpu.sync_copy(msg,
                        agg_sh.at[pl.ds(pl.multiple_of(row0 + k * C, 8), C)])
        return carry

    lax.fori_loop(0, RF, zcp, 0)
    pltpu.sync_copy(msg.at[pl.ds(0, RR)],
                    agg_sh.at[pl.ds(pl.multiple_of(row0 + RF * C, 8), RR)])

    @pl.when(sid == NS - 1)
    def _():
        pltpu.sync_copy(msg.at[pl.ds(0, TAILR)],
                        agg_sh.at[pl.ds(N - TAILR, TAILR)])

    plsc.subcore_barrier()

    lane = lax.iota(jnp.int32, L)
    cidv = jnp.full((L,), cid, jnp.int32)
    md = jnp.logical_and(lane == 1, cidv == 1)
    mc = jnp.logical_and(lane == 2, cidv == 1)
    vclmp = jnp.full((L,), VMAX - DLUT, jnp.float32)
    vlo = jnp.full((L,), -VMAX, jnp.float32)

    def batch(bi, carry):
        rb = sid * RPT2 + bi * NCHB
        pltpu.sync_copy(src_hbm.at[pl.ds(rb, NCHB)], src_b)
        pltpu.sync_copy(tgt_hbm.at[pl.ds(rb, NCHB)], tgt_b)
        pltpu.sync_copy(a_hbm.at[pl.ds(rb, NCHB)], a_b)
        pltpu.sync_copy(d_hbm.at[pl.ds(rb, NCHB)], d_b.at[pl.ds(0, NCHB)])
        for j in range(NCHB):
            for i in range(C // L):
                sl = pl.ds(i * L, L)
                ua_b[j, sl] = (w1r + w1r) * a_b[j, sl]
                s_ = src_b[j, sl]
                t_ = tgt_b[j, sl]
                sidx_b[j, sl] = s_ + s_ + cidv
                tidx_b[j, sl] = t_ + t_ + cidv

        gat = [None, None]
        sca = [None, None]
        gat[0] = (pltpu.async_copy(f2_hbm.at[sidx_b.at[0]], rows_s[0], sem_s),
                  pltpu.async_copy(f2_hbm.at[tidx_b.at[0]], rows_t[0], sem_t))
        for j in range(NCHB):
            p = j % 2
            gat[p][0].wait()
            gat[p][1].wait()
            if j + 1 < NCHB:
                q = 1 - p
                gat[q] = (
                    pltpu.async_copy(f2_hbm.at[sidx_b.at[j + 1]],
                                     rows_s[q], sem_s),
                    pltpu.async_copy(f2_hbm.at[tidx_b.at[j + 1]],
                                     rows_t[q], sem_t))
            if sca[p] is not None:
                sca[p].wait()
            rs, rt, mg = rows_s[p], rows_t[p], msgs[p]

            def egroup(g, ecarry, j=j, rs=rs, rt=rt, mg=mg):
                e0 = g * L
                ua16 = ua_b[j, pl.ds(e0, L)]
                de16 = d_b[j, pl.ds(e0, L)]
                for jj in range(L):
                    ua = jnp.full((L,), ua16[jj], jnp.float32)
                    for b in range(NBH):
                        xs = rs[e0 + jj, pl.ds(b * L, L)]
                        xt = rt[e0 + jj, pl.ds(b * L, L)]
                        ps = jnp.minimum(jnp.maximum(ua * xs, vlo), vclmp) \
                            * SLUT + (KLUT / 2.0 + 0.5)
                        pt = jnp.minimum(jnp.maximum(ua * xt, vlo), vclmp) \
                            * SLUT + (KLUT / 2.0 + 0.5)
                        m = (plsc.load_gather(lut_v,
                                               [ps.astype(jnp.int32)])
                             + plsc.load_gather(lut_v,
                                                [pt.astype(jnp.int32)]))
                        if b == 3:
                            de = jnp.full((L,), de16[jj], jnp.float32)
                            m = jnp.where(md, de, m)
                            m = jnp.where(mc, jnp.float32(1.0), m)
                        mg[e0 + jj, pl.ds(b * L, L)] = m
                return ecarry

            lax.fori_loop(0, C // L, egroup, 0)
            sca[p] = pltpu.async_copy(mg, agg_sh.at[tgt_b.at[j]], ssems[p],
                                      add=True)
        sca[0].wait()
        sca[1].wait()
        return carry

    lax.fori_loop(0, NBATCH, batch, 0)
    plsc.subcore_barrier()

    @pl.when(cid == 0)
    def _():
        def wout(k, carry):
            r = pl.multiple_of(row0 + k * C, 8)
            pltpu.sync_copy(agg_sh.at[pl.ds(r, C)], agg0_hbm.at[pl.ds(r, C)])
            return carry

        lax.fori_loop(0, RF, wout, 0)
        rl = pl.multiple_of(row0 + RF * C, 8)
        pltpu.sync_copy(agg_sh.at[pl.ds(rl, RR)], agg0_hbm.at[pl.ds(rl, RR)])

        @pl.when(sid == NS - 1)
        def _():
            pltpu.sync_copy(agg_sh.at[pl.ds(N - TAILR, TAILR)],
                            agg0_hbm.at[pl.ds(N - TAILR, TAILR)])

    @pl.when(cid == 1)
    def _():
        def wout(k, carry):
            r = pl.multiple_of(row0 + k * C, 8)
            pltpu.sync_copy(agg_sh.at[pl.ds(r, C)], agg1_hbm.at[pl.ds(r, C)])
            return carry

        lax.fori_loop(0, RF, wout, 0)
        rl = pl.multiple_of(row0 + RF * C, 8)
        pltpu.sync_copy(agg_sh.at[pl.ds(rl, RR)], agg1_hbm.at[pl.ds(rl, RR)])

        @pl.when(sid == NS - 1)
        def _():
            pltpu.sync_copy(agg_sh.at[pl.ds(N - TAILR, TAILR)],
                            agg1_hbm.at[pl.ds(N - TAILR, TAILR)])


_sc_call = pl.kernel(
    _sc_body,
    out_type=(jax.ShapeDtypeStruct((N, DH), jnp.float32),
              jax.ShapeDtypeStruct((N, DH), jnp.float32)),
    mesh=plsc.VectorSubcoreMesh(core_axis_name="c", subcore_axis_name="s"),
    compiler_params=pltpu.CompilerParams(use_tc_tiling_on_sc=False,
                                         needs_layout_passes=False),
    scratch_types=[
        pltpu.VMEM((NCHB, C), jnp.int32),
        pltpu.VMEM((NCHB, C), jnp.int32),
        pltpu.VMEM((NCHB, C), jnp.int32),
        pltpu.VMEM((NCHB, C), jnp.int32),
        pltpu.VMEM((NCHB, C), jnp.float32),
        pltpu.VMEM((NCHB + 1, C), jnp.float32),
        pltpu.VMEM((NCHB + 1, C), jnp.float32),
        pltpu.VMEM((L,), jnp.float32),
        pltpu.VMEM((KLUT,), jnp.float32),
        pltpu.VMEM((C, DH), jnp.float32),
        pltpu.VMEM((C, DH), jnp.float32),
        pltpu.VMEM((C, DH), jnp.float32),
        pltpu.VMEM((C, DH), jnp.float32),
        pltpu.VMEM((C, DH), jnp.float32),
        pltpu.VMEM((C, DH), jnp.float32),
        pltpu.VMEM_SHARED((N, DH), jnp.float32),
        pltpu.SemaphoreType.DMA,
        pltpu.SemaphoreType.DMA,
        pltpu.SemaphoreType.DMA,
        pltpu.SemaphoreType.DMA,
    ],
)


BROWS = 2000


def _tc_body(f_ref, a0_ref, a1_ref, W1_ref, b1_ref, W2_ref, b2_ref,
             W3_ref, b3_ref, o_ref):
    agg = jnp.concatenate([a0_ref[...], a1_ref[...]], axis=1)
    col = lax.broadcasted_iota(jnp.int32, agg.shape, 1)
    aggm = jnp.where(col < D, agg, 0.0)
    nd = agg[:, D:D + 1]
    ncnt = agg[:, D + 1:D + 2]
    f = f_ref[...]
    f_inv = jnp.sqrt(jnp.sum(f * f, axis=1, keepdims=True))
    msg_inv = jnp.sqrt(jnp.sum(aggm * aggm, axis=1, keepdims=True))
    avg = nd / (ncnt + 1e-8)
    psi = jnp.concatenate([f_inv, msg_inv, avg], axis=1)
    h = jax.nn.relu(jnp.dot(psi, W1_ref[...].T,
                            preferred_element_type=jnp.float32) + b1_ref[...])
    h = jax.nn.relu(jnp.dot(h, W2_ref[...].T,
                            preferred_element_type=jnp.float32) + b2_ref[...])
    gate = jax.nn.sigmoid(jnp.sum(h * W3_ref[...], axis=1, keepdims=True)
                          + b3_ref[0, 0])
    o_ref[...] = f + gate * aggm


def _tc_call(fpad, a0, a1, W1, b1, W2, b2, W3, b3):
    full = lambda shape: pl.BlockSpec(shape, lambda i: (0, 0))
    return pl.pallas_call(
        _tc_body,
        grid=(N // BROWS,),
        in_specs=[
            pl.BlockSpec((BROWS, DP), lambda i: (i, 0)),
            pl.BlockSpec((BROWS, DH), lambda i: (i, 0)),
            pl.BlockSpec((BROWS, DH), lambda i: (i, 0)),
            full((64, 3)), full((1, 64)),
            full((32, 64)), full((1, 32)),
            full((1, 32)),
            pl.BlockSpec(memory_space=pltpu.SMEM),
        ],
        out_specs=pl.BlockSpec((BROWS, DP), lambda i: (i, 0)),
        out_shape=jax.ShapeDtypeStruct((N, DP), jnp.float32),
    )(fpad, a0, a1, W1, b1, W2, b2, W3, b3)


@jax.jit
def kernel(edge_index, f, d, a, w1, w2, W1, b1, W2, b2, W3, b3):
    src = edge_index[0].astype(jnp.int32).reshape(ER, C)
    tgt = edge_index[1].astype(jnp.int32).reshape(ER, C)
    fpad = jnp.pad(f, ((0, 0), (0, DP - D)))
    f2 = fpad.reshape(N, 2, DH).reshape(2 * N, DH)
    a2 = a[:, 0].reshape(ER, C)
    d2 = d[:, 0].reshape(ER, C)
    w1b = jnp.full((L,), w1[0], jnp.float32)
    lut = _lut_call(w2.reshape(1, 1)).reshape(KLUT)
    agg0, agg1 = _sc_call(f2, src, tgt, a2, d2, w1b, lut)
    outp = _tc_call(fpad, agg0, agg1, W1, b1.reshape(1, 64),
                    W2, b2.reshape(1, 32), W3, b3.reshape(1, 1))
    return outp[:, :D]
